# baseline (device time: 317611 ns/iter reference)
import jax
import jax.numpy as jnp
from jax import lax
from jax.experimental import pallas as pl
from jax.experimental.pallas import tpu as pltpu

N_DEV = 4
SQ = 2048
SKV = 2048
D_MODEL = 1024
H_PER = 8
DH = 128
SCALE = 0.08838834764831843
QB = 256
BAND = 512


def _body(x_ref, w_ref, k_hbm, v_hbm, out_ref, comm, q_ref, ctx_ref,
          k_vm, v_vm, send_sems, recv_sems, dma_sems):
    my = lax.axis_index("i")
    right = lax.rem(my + 1, N_DEV)
    left = lax.rem(my + 3, N_DEV)

    barrier = pltpu.get_barrier_semaphore()
    for nbr in (left, right):
        pl.semaphore_signal(barrier, inc=1, device_id=(nbr,),
                            device_id_type=pl.DeviceIdType.MESH)
    pl.semaphore_wait(barrier, 2)

    comm[0, :, :] = w_ref[:, :]
    out_ref[0] = jnp.zeros((SQ, D_MODEL), jnp.float32)

    def compute_block(r):
        origin = lax.rem(my - r + N_DEV, N_DEV)
        hd0 = origin * H_PER
        ck = pltpu.make_async_copy(
            k_hbm.at[pl.ds(hd0, H_PER)], k_vm, dma_sems.at[0])
        cv = pltpu.make_async_copy(
            v_hbm.at[pl.ds(hd0, H_PER)], v_vm, dma_sems.at[1])
        ck.start()
        cv.start()

        q_ref[...] = (
            jnp.dot(x_ref[...], comm[r, 0:D_MODEL, :],
                    preferred_element_type=jnp.float32)
            * SCALE
        ).astype(jnp.bfloat16)
        ck.wait()
        cv.wait()

        def head(h, carry):
            def do_qb(r0, nrows, pieces):
                qh = q_ref[pl.ds(r0, nrows), pl.ds(h * DH, DH)]
                ss = []
                for lo, width, mode in pieces:
                    kp = k_vm[h, pl.ds(lo, width), :]
                    s = lax.dot_general(
                        qh, kp, (((1,), (1,)), ((), ())),
                        preferred_element_type=jnp.float32)
                    if mode == "none":
                        ss.append(s)
                        continue
                    ci = lax.broadcasted_iota(jnp.int32, (nrows, width), 1)
                    if mode == "glob":
                        mask = ci < 32
                    else:
                        qi = r0 + lax.broadcasted_iota(
                            jnp.int32, (nrows, width), 0)
                        ki = lo + ci
                        mask = jnp.abs(qi - ki) <= 128
                        if mode == "full":
                            mask = mask | (ki < 32) | (qi < 32)
                    ss.append(jnp.where(mask, s, -1e9))
                m = ss[0].max(axis=1, keepdims=True)
                for s in ss[1:]:
                    m = jnp.maximum(m, s.max(axis=1, keepdims=True))
                es = [jnp.exp(s - m) for s in ss]
                denom = es[0].sum(axis=1, keepdims=True)
                for e in es[1:]:
                    denom = denom + e.sum(axis=1, keepdims=True)
                acc = None
                for e, (lo, width, _) in zip(es, pieces):
                    vp = v_vm[h, pl.ds(lo, width), :]
                    pv = jnp.dot(e.astype(jnp.bfloat16), vp,
                                 preferred_element_type=jnp.float32)
                    acc = pv if acc is None else acc + pv
                ctx_ref[pl.ds(r0, nrows), pl.ds(h * DH, DH)] = (
                    acc / denom).astype(jnp.bfloat16)

            do_qb(0, 32, [(0, SKV, "none")])
            do_qb(32, 96, [(0, 256, "full")])
            do_qb(128, 128, [(0, 384, "full")])

            def qb_loop(qb, c):
                r0 = qb * QB
                lo = jnp.minimum(r0 - 128, SKV - BAND)
                do_qb(r0, QB, [(0, 128, "glob"), (lo, BAND, "band")])
                return c
            lax.fori_loop(1, SQ // QB, qb_loop, 0)
            return carry

        import os as _os
        if not _os.environ.get("SKIP_ATTN"):
            lax.fori_loop(0, H_PER, head, 0)

        part = jnp.dot(ctx_ref[...], comm[r, D_MODEL:2 * D_MODEL, :],
                       preferred_element_type=jnp.float32)
        out_ref[0] = out_ref[0] + part

    def hop(h, c):
        rdma = pltpu.make_async_remote_copy(
            src_ref=comm.at[h],
            dst_ref=comm.at[h + 1],
            send_sem=send_sems.at[h],
            recv_sem=recv_sems.at[h],
            device_id=(right,),
            device_id_type=pl.DeviceIdType.MESH,
        )
        rdma.start()
        compute_block(h)
        rdma.wait()
        return c

    lax.fori_loop(0, N_DEV - 1, hop, 0)
    compute_block(N_DEV - 1)


def kernel(x, Wq, K_ext, V_ext, Wo):
    my = lax.axis_index("i")
    xb = x[0].astype(jnp.bfloat16)
    w_my = jnp.concatenate(
        [Wq.astype(jnp.bfloat16), Wo.astype(jnp.bfloat16)], axis=0
    )
    kb = jnp.transpose(
        lax.dynamic_index_in_dim(K_ext, my, 0, keepdims=False), (1, 0, 2)
    ).astype(jnp.bfloat16)
    vb = jnp.transpose(
        lax.dynamic_index_in_dim(V_ext, my, 0, keepdims=False), (1, 0, 2)
    ).astype(jnp.bfloat16)

    return pl.pallas_call(
        _body,
        out_shape=jax.ShapeDtypeStruct((1, SQ, D_MODEL), jnp.float32),
        in_specs=[
            pl.BlockSpec(memory_space=pltpu.VMEM),
            pl.BlockSpec(memory_space=pltpu.VMEM),
            pl.BlockSpec(memory_space=pl.ANY),
            pl.BlockSpec(memory_space=pl.ANY),
        ],
        out_specs=pl.BlockSpec(memory_space=pltpu.VMEM),
        scratch_shapes=[
            pltpu.VMEM((N_DEV, 2 * D_MODEL, D_MODEL), jnp.bfloat16),
            pltpu.VMEM((SQ, D_MODEL), jnp.bfloat16),
            pltpu.VMEM((SQ, D_MODEL), jnp.bfloat16),
            pltpu.VMEM((H_PER, SKV, DH), jnp.bfloat16),
            pltpu.VMEM((H_PER, SKV, DH), jnp.bfloat16),
            pltpu.SemaphoreType.DMA((N_DEV - 1,)),
            pltpu.SemaphoreType.DMA((N_DEV - 1,)),
            pltpu.SemaphoreType.DMA((2,)),
        ],
        compiler_params=pltpu.CompilerParams(
            collective_id=0, vmem_limit_bytes=100 * 1024 * 1024),
    )(xb, w_my, kb, vb)


# device time: 312865 ns/iter; 1.0152x vs baseline; 1.0152x over previous
import jax
import jax.numpy as jnp
from jax import lax
from jax.experimental import pallas as pl
from jax.experimental.pallas import tpu as pltpu

N_DEV = 4
SQ = 2048
SKV = 2048
D_MODEL = 1024
H_PER = 8
DH = 128
SCALE = 0.08838834764831843
QB = 256
BAND = 512


def _body(x_ref, w_ref, k_hbm, v_hbm, out_ref, comm, q_ref, ctx_ref,
          k_vm, v_vm, send_sems, recv_sems, dma_sems):
    my = lax.axis_index("i")
    right = lax.rem(my + 1, N_DEV)
    left = lax.rem(my + 3, N_DEV)

    barrier = pltpu.get_barrier_semaphore()
    for nbr in (left, right):
        pl.semaphore_signal(barrier, inc=1, device_id=(nbr,),
                            device_id_type=pl.DeviceIdType.MESH)
    pl.semaphore_wait(barrier, 2)

    comm[0, :, :] = w_ref[0:D_MODEL, :]
    comm[1, :, :] = w_ref[D_MODEL:2 * D_MODEL, :]
    out_ref[0] = jnp.zeros((SQ, D_MODEL), jnp.float32)

    def compute_block(r):
        origin = lax.rem(my - r + N_DEV, N_DEV)
        hd0 = origin * H_PER
        ck = pltpu.make_async_copy(
            k_hbm.at[pl.ds(hd0, H_PER)], k_vm, dma_sems.at[0])
        cv = pltpu.make_async_copy(
            v_hbm.at[pl.ds(hd0, H_PER)], v_vm, dma_sems.at[1])
        ck.start()
        cv.start()

        q_ref[...] = (
            jnp.dot(x_ref[...], comm[2 * r],
                    preferred_element_type=jnp.float32)
            * SCALE
        ).astype(jnp.bfloat16)
        ck.wait()
        cv.wait()

        def head(h, carry):
            def do_qb(r0, nrows, pieces):
                qh = q_ref[pl.ds(r0, nrows), pl.ds(h * DH, DH)]
                ss = []
                for lo, width, mode in pieces:
                    kp = k_vm[h, pl.ds(lo, width), :]
                    s = lax.dot_general(
                        qh, kp, (((1,), (1,)), ((), ())),
                        preferred_element_type=jnp.float32)
                    if mode == "none":
                        ss.append(s)
                        continue
                    ci = lax.broadcasted_iota(jnp.int32, (nrows, width), 1)
                    if mode == "glob":
                        mask = ci < 32
                    else:
                        qi = r0 + lax.broadcasted_iota(
                            jnp.int32, (nrows, width), 0)
                        ki = lo + ci
                        mask = jnp.abs(qi - ki) <= 128
                        if mode == "full":
                            mask = mask | (ki < 32) | (qi < 32)
                    ss.append(jnp.where(mask, s, -1e9))
                m = ss[0].max(axis=1, keepdims=True)
                for s in ss[1:]:
                    m = jnp.maximum(m, s.max(axis=1, keepdims=True))
                es = [jnp.exp(s - m) for s in ss]
                denom = es[0].sum(axis=1, keepdims=True)
                for e in es[1:]:
                    denom = denom + e.sum(axis=1, keepdims=True)
                acc = None
                for e, (lo, width, _) in zip(es, pieces):
                    vp = v_vm[h, pl.ds(lo, width), :]
                    pv = jnp.dot(e.astype(jnp.bfloat16), vp,
                                 preferred_element_type=jnp.float32)
                    acc = pv if acc is None else acc + pv
                ctx_ref[pl.ds(r0, nrows), pl.ds(h * DH, DH)] = (
                    acc / denom).astype(jnp.bfloat16)

            do_qb(0, 32, [(0, SKV, "none")])
            do_qb(32, 96, [(0, 256, "full")])
            do_qb(128, 128, [(0, 384, "full")])

            def qb_loop(qb, c):
                r0 = qb * QB
                lo = jnp.minimum(r0 - 128, SKV - BAND)
                do_qb(r0, QB, [(0, 128, "glob"), (lo, BAND, "band")])
                return c
            lax.fori_loop(1, SQ // QB, qb_loop, 0)
            return carry

        import os as _os
        if not _os.environ.get("SKIP_ATTN"):
            lax.fori_loop(0, H_PER, head, 0)

        part = jnp.dot(ctx_ref[...], comm[2 * r + 1],
                       preferred_element_type=jnp.float32)
        out_ref[0] = out_ref[0] + part

    r1 = pltpu.make_async_remote_copy(
        src_ref=comm.at[pl.ds(0, 2)], dst_ref=comm.at[pl.ds(2, 2)],
        send_sem=send_sems.at[0], recv_sem=recv_sems.at[0],
        device_id=(right,), device_id_type=pl.DeviceIdType.MESH,
    )
    l1 = pltpu.make_async_remote_copy(
        src_ref=comm.at[pl.ds(0, 2)], dst_ref=comm.at[pl.ds(6, 2)],
        send_sem=send_sems.at[1], recv_sem=recv_sems.at[1],
        device_id=(left,), device_id_type=pl.DeviceIdType.MESH,
    )
    r1.start()
    l1.start()
    compute_block(0)
    r1.wait()
    l1.wait()

    r2 = pltpu.make_async_remote_copy(
        src_ref=comm.at[2], dst_ref=comm.at[4],
        send_sem=send_sems.at[2], recv_sem=recv_sems.at[2],
        device_id=(right,), device_id_type=pl.DeviceIdType.MESH,
    )
    l2 = pltpu.make_async_remote_copy(
        src_ref=comm.at[7], dst_ref=comm.at[5],
        send_sem=send_sems.at[3], recv_sem=recv_sems.at[3],
        device_id=(left,), device_id_type=pl.DeviceIdType.MESH,
    )
    r2.start()
    l2.start()

    def mid(i, c):
        compute_block(1 + 2 * i)
        return c

    lax.fori_loop(0, 2, mid, 0)
    r2.wait()
    l2.wait()
    compute_block(2)


def kernel(x, Wq, K_ext, V_ext, Wo):
    my = lax.axis_index("i")
    xb = x[0].astype(jnp.bfloat16)
    w_my = jnp.concatenate(
        [Wq.astype(jnp.bfloat16), Wo.astype(jnp.bfloat16)], axis=0
    )
    kb = jnp.transpose(
        lax.dynamic_index_in_dim(K_ext, my, 0, keepdims=False), (1, 0, 2)
    ).astype(jnp.bfloat16)
    vb = jnp.transpose(
        lax.dynamic_index_in_dim(V_ext, my, 0, keepdims=False), (1, 0, 2)
    ).astype(jnp.bfloat16)

    return pl.pallas_call(
        _body,
        out_shape=jax.ShapeDtypeStruct((1, SQ, D_MODEL), jnp.float32),
        in_specs=[
            pl.BlockSpec(memory_space=pltpu.VMEM),
            pl.BlockSpec(memory_space=pltpu.VMEM),
            pl.BlockSpec(memory_space=pl.ANY),
            pl.BlockSpec(memory_space=pl.ANY),
        ],
        out_specs=pl.BlockSpec(memory_space=pltpu.VMEM),
        scratch_shapes=[
            pltpu.VMEM((2 * N_DEV, D_MODEL, D_MODEL), jnp.bfloat16),
            pltpu.VMEM((SQ, D_MODEL), jnp.bfloat16),
            pltpu.VMEM((SQ, D_MODEL), jnp.bfloat16),
            pltpu.VMEM((H_PER, SKV, DH), jnp.bfloat16),
            pltpu.VMEM((H_PER, SKV, DH), jnp.bfloat16),
            pltpu.SemaphoreType.DMA((4,)),
            pltpu.SemaphoreType.DMA((4,)),
            pltpu.SemaphoreType.DMA((2,)),
        ],
        compiler_params=pltpu.CompilerParams(
            collective_id=0, vmem_limit_bytes=100 * 1024 * 1024),
    )(xb, w_my, kb, vb)


# device time: 244558 ns/iter; 1.2987x vs baseline; 1.2793x over previous
import jax
import jax.numpy as jnp
from jax import lax
from jax.experimental import pallas as pl
from jax.experimental.pallas import tpu as pltpu

N_DEV = 4
SQ = 2048
SKV = 2048
D_MODEL = 1024
H_PER = 8
DH = 128
SCALE = 0.08838834764831843
QB = 256
BAND = 512


def _body(x_ref, w_ref, k_hbm, v_hbm, out_ref, comm, q_ref, ctx_ref,
          k_vm, v_vm, send_sems, recv_sems, dma_sems):
    my = lax.axis_index("i")
    right = lax.rem(my + 1, N_DEV)
    left = lax.rem(my + 3, N_DEV)

    barrier = pltpu.get_barrier_semaphore()
    for nbr in (left, right):
        pl.semaphore_signal(barrier, inc=1, device_id=(nbr,),
                            device_id_type=pl.DeviceIdType.MESH)
    pl.semaphore_wait(barrier, 2)

    out_ref[0] = jnp.zeros((SQ, D_MODEL), jnp.bfloat16)

    def compute_block(r, wq, wo):
        origin = lax.rem(my - r + N_DEV, N_DEV)
        hd0 = origin * H_PER
        ck = pltpu.make_async_copy(
            k_hbm.at[pl.ds(hd0, H_PER)], k_vm, dma_sems.at[0])
        cv = pltpu.make_async_copy(
            v_hbm.at[pl.ds(hd0, H_PER)], v_vm, dma_sems.at[1])
        ck.start()
        cv.start()

        for i in range(2):
            rows = slice(i * SQ // 2, (i + 1) * SQ // 2)
            qval = (
                jnp.dot(x_ref[rows, :], wq,
                        preferred_element_type=jnp.float32)
                * SCALE
            ).astype(jnp.bfloat16)
            for h in range(H_PER):
                q_ref[h, rows, :] = qval[:, h * DH:(h + 1) * DH]
        ck.wait()
        cv.wait()

        def do_qb(r0, nrows, pieces):
            qh = q_ref[:, pl.ds(r0, nrows), :]
            ss = []
            for lo, width, mode in pieces:
                kp = k_vm[:, pl.ds(lo, width), :]
                s = lax.dot_general(
                    qh, kp, (((2,), (2,)), ((0,), (0,))),
                    preferred_element_type=jnp.float32)
                if mode == "none":
                    ss.append(s)
                    continue
                ci = lax.broadcasted_iota(jnp.int32, (nrows, width), 1)
                if mode == "glob":
                    mask = ci < 32
                else:
                    qi = r0 + lax.broadcasted_iota(
                        jnp.int32, (nrows, width), 0)
                    ki = lo + ci
                    mask = jnp.abs(qi - ki) <= 128
                    if mode == "full":
                        mask = mask | (ki < 32) | (qi < 32)
                ss.append(jnp.where(mask[None, :, :], s, -1e9))
            m = ss[0].max(axis=-1, keepdims=True)
            for s in ss[1:]:
                m = jnp.maximum(m, s.max(axis=-1, keepdims=True))
            es = [jnp.exp(s - m) for s in ss]
            denom = es[0].sum(axis=-1, keepdims=True)
            for e in es[1:]:
                denom = denom + e.sum(axis=-1, keepdims=True)
            acc = None
            for e, (lo, width, _) in zip(es, pieces):
                vp = v_vm[:, pl.ds(lo, width), :]
                pv = lax.dot_general(
                    e.astype(jnp.bfloat16), vp,
                    (((2,), (1,)), ((0,), (0,))),
                    preferred_element_type=jnp.float32)
                acc = pv if acc is None else acc + pv
            q_ref[:, pl.ds(r0, nrows), :] = (
                acc / denom).astype(jnp.bfloat16)

        def dense_head(h, c):
            qh = q_ref[h, 0:32, :]
            s = lax.dot_general(
                qh, k_vm[h], (((1,), (1,)), ((), ())),
                preferred_element_type=jnp.float32)
            m = s.max(axis=-1, keepdims=True)
            e = jnp.exp(s - m)
            denom = e.sum(axis=-1, keepdims=True)
            pv = jnp.dot(e.astype(jnp.bfloat16), v_vm[h],
                         preferred_element_type=jnp.float32)
            q_ref[h, 0:32, :] = (pv / denom).astype(jnp.bfloat16)
            return c
        lax.fori_loop(0, H_PER, dense_head, 0)
        do_qb(32, 96, [(0, 256, "full")])
        do_qb(128, 128, [(0, 384, "full")])

        def qb_loop(qb, c):
            r0 = qb * QB
            lo = jnp.minimum(r0 - 128, SKV - BAND)
            do_qb(r0, QB, [(0, 128, "glob"), (lo, BAND, "band")])
            return c
        lax.fori_loop(1, SQ // QB, qb_loop, 0)

        for h in range(H_PER):
            ctx_ref[:, h * DH:(h + 1) * DH] = q_ref[h]
        for i in range(2):
            rows = slice(i * SQ // 2, (i + 1) * SQ // 2)
            part = jnp.dot(ctx_ref[rows, :], wo,
                           preferred_element_type=jnp.float32)
            out_ref[0, rows, :] = (
                out_ref[0, rows, :].astype(jnp.float32) + part
            ).astype(jnp.bfloat16)

    r1 = pltpu.make_async_remote_copy(
        src_ref=w_ref, dst_ref=comm.at[pl.ds(0, 2)],
        send_sem=send_sems.at[0], recv_sem=recv_sems.at[0],
        device_id=(right,), device_id_type=pl.DeviceIdType.MESH,
    )
    l1 = pltpu.make_async_remote_copy(
        src_ref=w_ref, dst_ref=comm.at[pl.ds(4, 2)],
        send_sem=send_sems.at[1], recv_sem=recv_sems.at[1],
        device_id=(left,), device_id_type=pl.DeviceIdType.MESH,
    )
    r1.start()
    l1.start()
    compute_block(0, w_ref[0], w_ref[1])
    r1.wait()
    l1.wait()

    r2 = pltpu.make_async_remote_copy(
        src_ref=comm.at[0], dst_ref=comm.at[2],
        send_sem=send_sems.at[2], recv_sem=recv_sems.at[2],
        device_id=(right,), device_id_type=pl.DeviceIdType.MESH,
    )
    l2 = pltpu.make_async_remote_copy(
        src_ref=comm.at[5], dst_ref=comm.at[3],
        send_sem=send_sems.at[3], recv_sem=recv_sems.at[3],
        device_id=(left,), device_id_type=pl.DeviceIdType.MESH,
    )
    r2.start()
    l2.start()

    def mid(i, c):
        r = 1 + 2 * i
        compute_block(r, comm[2 * r - 2], comm[2 * r - 1])
        return c

    lax.fori_loop(0, 2, mid, 0)
    r2.wait()
    l2.wait()
    compute_block(2, comm[2], comm[3])


def kernel(x, Wq, K_ext, V_ext, Wo):
    my = lax.axis_index("i")
    xb = x[0].astype(jnp.bfloat16)
    w_my = jnp.stack(
        [Wq.astype(jnp.bfloat16), Wo.astype(jnp.bfloat16)], axis=0
    )
    kb = jnp.transpose(
        lax.dynamic_index_in_dim(K_ext, my, 0, keepdims=False), (1, 0, 2)
    ).astype(jnp.bfloat16)
    vb = jnp.transpose(
        lax.dynamic_index_in_dim(V_ext, my, 0, keepdims=False), (1, 0, 2)
    ).astype(jnp.bfloat16)

    return pl.pallas_call(
        _body,
        out_shape=jax.ShapeDtypeStruct((1, SQ, D_MODEL), jnp.bfloat16),
        in_specs=[
            pl.BlockSpec(memory_space=pltpu.VMEM),
            pl.BlockSpec(memory_space=pltpu.VMEM),
            pl.BlockSpec(memory_space=pl.ANY),
            pl.BlockSpec(memory_space=pl.ANY),
        ],
        out_specs=pl.BlockSpec(memory_space=pltpu.VMEM),
        scratch_shapes=[
            pltpu.VMEM((6, D_MODEL, D_MODEL), jnp.bfloat16),
            pltpu.VMEM((H_PER, SQ, DH), jnp.bfloat16),
            pltpu.VMEM((SQ, D_MODEL), jnp.bfloat16),
            pltpu.VMEM((H_PER, SKV, DH), jnp.bfloat16),
            pltpu.VMEM((H_PER, SKV, DH), jnp.bfloat16),
            pltpu.SemaphoreType.DMA((4,)),
            pltpu.SemaphoreType.DMA((4,)),
            pltpu.SemaphoreType.DMA((2,)),
        ],
        compiler_params=pltpu.CompilerParams(
            collective_id=0, vmem_limit_bytes=100 * 1024 * 1024),
    )(xb, w_my, kb, vb)
